# baseline (speedup 1.0000x reference)
import functools

import jax
import jax.numpy as jnp
from jax import lax
from jax.experimental import pallas as pl
from jax.experimental.pallas import tpu as pltpu
from jax.experimental.pallas import tpu_sc as plsc

NC = 2
NS = 16
LANES = 16
NW = NC * NS

L = 16384
D = 64
NTAGS = 16
PER_W = L // NW
CHUNK = 128
NCHUNK = PER_W // CHUNK
DV = D // LANES


def _cbow_sc(words3, table, wt, b):
    mesh = plsc.VectorSubcoreMesh(core_axis_name="c", subcore_axis_name="s")

    @functools.partial(
        pl.kernel,
        out_type=jax.ShapeDtypeStruct((NC, NTAGS), jnp.float32),
        mesh=mesh,
        scratch_types=[
            pltpu.VMEM((NCHUNK, CHUNK), jnp.int32),
            pltpu.VMEM((NCHUNK, CHUNK, D), jnp.float32),
            pltpu.VMEM((D,), jnp.float32),
            pltpu.VMEM((NS, D), jnp.float32),
            pltpu.VMEM((D, NTAGS), jnp.float32),
            pltpu.VMEM((NTAGS,), jnp.float32),
            pltpu.VMEM((NTAGS,), jnp.float32),
            pltpu.VMEM_SHARED((NS, D), jnp.float32),
            pltpu.SemaphoreType.DMA,
        ],
        compiler_params=pltpu.CompilerParams(use_tc_tiling_on_sc=False),
    )
    def k(words_hbm, table_hbm, wt_hbm, b_hbm, out_hbm,
          idx_v, rows_v, sum_v, allv, wt_v, b_v, out_v, shared, sem):
        cid = lax.axis_index("c")
        sid = lax.axis_index("s")
        wid = cid * NS + sid

        pltpu.sync_copy(words_hbm.at[wid], idx_v)
        copies = [
            pltpu.async_copy(table_hbm.at[idx_v.at[j]], rows_v.at[j], sem)
            for j in range(NCHUNK)
        ]
        for cp in copies:
            cp.wait()

        acc = tuple(jnp.zeros((LANES,), jnp.float32) for _ in range(DV))
        for j in range(NCHUNK):
            def body(i, carry, j=j):
                return tuple(
                    carry[t] + rows_v[j, i, pl.ds(t * LANES, LANES)]
                    for t in range(DV)
                )
            acc = lax.fori_loop(0, CHUNK, body, acc)
        for t in range(DV):
            sum_v[pl.ds(t * LANES, LANES)] = acc[t]

        pltpu.sync_copy(sum_v, shared.at[sid])
        plsc.subcore_barrier()

        @pl.when(sid == 0)
        def _():
            pltpu.sync_copy(shared, allv)
            pltpu.sync_copy(wt_hbm, wt_v)
            pltpu.sync_copy(b_hbm, b_v)
            tot = tuple(jnp.zeros((LANES,), jnp.float32) for _ in range(DV))
            for s in range(NS):
                tot = tuple(
                    tot[t] + allv[s, pl.ds(t * LANES, LANES)]
                    for t in range(DV)
                )
            bias = b_v[...]
            zero = jnp.zeros((NTAGS,), jnp.float32)
            out = jnp.where(cid == 0, bias, zero)
            for t in range(DV):
                blk = tot[t]
                for lane in range(LANES):
                    out = out + blk[lane] * wt_v[t * LANES + lane]
            out_v[...] = out
            pltpu.sync_copy(out_v, out_hbm.at[cid])

    return k(words3, table, wt, b)


def kernel(words, embedding, W, b):
    words3 = words.astype(jnp.int32).reshape(NW, NCHUNK, CHUNK)
    wt = W.T
    partials = _cbow_sc(words3, embedding, wt, b)
    return jnp.sum(partials, axis=0).reshape(1, NTAGS)


# speedup vs baseline: 4.1337x; 4.1337x over previous
import functools

import jax
import jax.numpy as jnp
from jax import lax
from jax.experimental import pallas as pl
from jax.experimental.pallas import tpu as pltpu
from jax.experimental.pallas import tpu_sc as plsc

NS = 16
LANES = 16

L = 16384
D = 64
NTAGS = 16
VOCAB = 1_000_000
CPAD = 1_048_576
PER_T = L // NS
CHUNK = CPAD // NS

BLK = 8192
NBLK = (VOCAB + BLK - 1) // BLK
SUBS = BLK // 128
TAIL_SUBS = (VOCAB - (NBLK - 1) * BLK + 127) // 128


def _count_sc(words3):
    mesh = plsc.VectorSubcoreMesh(
        core_axis_name="c", subcore_axis_name="s", num_cores=1
    )

    @functools.partial(
        pl.kernel,
        out_type=jax.ShapeDtypeStruct((CPAD,), jnp.float32),
        mesh=mesh,
        scratch_types=[
            pltpu.VMEM((PER_T // 128, 128), jnp.int32),
            pltpu.VMEM((CHUNK // 4,), jnp.float32),
            pltpu.VMEM((128,), jnp.float32),
            pltpu.VMEM_SHARED((CPAD,), jnp.float32),
        ],
    )
    def k(words_hbm, cnt_hbm, idx_v, zeros_v, ones_v, cnt_s):
        sid = lax.axis_index("s")

        z16 = jnp.zeros((LANES,), jnp.float32)
        o16 = jnp.ones((LANES,), jnp.float32)

        def zbody(i, _):
            zeros_v[pl.ds(i * LANES, LANES)] = z16
            return 0

        lax.fori_loop(0, (CHUNK // 4) // LANES, zbody, 0)
        for i in range(128 // LANES):
            ones_v[pl.ds(i * LANES, LANES)] = o16

        pltpu.sync_copy(words_hbm.at[sid], idx_v)
        for q in range(4):
            pltpu.sync_copy(
                zeros_v, cnt_s.at[pl.ds(sid * CHUNK + q * (CHUNK // 4),
                                        CHUNK // 4)]
            )
        plsc.subcore_barrier()

        for j in range(PER_T // 128):
            pltpu.sync_copy(ones_v, cnt_s.at[idx_v.at[j]], add=True)
        plsc.subcore_barrier()

        pltpu.sync_copy(
            cnt_s.at[pl.ds(sid * CHUNK, CHUNK)],
            cnt_hbm.at[pl.ds(sid * CHUNK, CHUNK)],
        )

    return k(words3)


def _matvec_tc(table_t, cnt2, wt, b2):
    def body(tbl_ref, cnt_ref, wt_ref, b_ref, out_ref, acc_ref):
        j = pl.program_id(0)

        @pl.when(j == 0)
        def _():
            acc_ref[...] = jnp.zeros((D, 128), jnp.float32)

        def accumulate(nsubs, mask_tail):
            acc = acc_ref[...]
            for sub in range(nsubs):
                t = tbl_ref[:, sub * 128:(sub + 1) * 128]
                if mask_tail:
                    base = (NBLK - 1) * BLK + sub * 128
                    ok = base + lax.iota(jnp.int32, 128) < VOCAB
                    t = jnp.where(ok[None, :], t, 0.0)
                row = cnt_ref[sub * 128:(sub + 1) * 128]
                acc = acc + t * row[None, :]
            acc_ref[...] = acc

        @pl.when(j < NBLK - 1)
        def _():
            accumulate(SUBS, False)

        @pl.when(j == NBLK - 1)
        def _():
            accumulate(TAIL_SUBS, True)
            emb = jnp.sum(acc_ref[...], axis=1)
            logits = jnp.sum(wt_ref[...] * emb[:, None], axis=0)
            out_ref[...] = logits[None, :] + b_ref[...]

        return

    return pl.pallas_call(
        body,
        grid=(NBLK,),
        in_specs=[
            pl.BlockSpec((D, BLK), lambda j: (0, j)),
            pl.BlockSpec((BLK,), lambda j: (j,)),
            pl.BlockSpec((D, NTAGS), lambda j: (0, 0)),
            pl.BlockSpec((1, NTAGS), lambda j: (0, 0)),
        ],
        out_specs=pl.BlockSpec((1, NTAGS), lambda j: (0, 0)),
        out_shape=jax.ShapeDtypeStruct((1, NTAGS), jnp.float32),
        scratch_shapes=[pltpu.VMEM((D, 128), jnp.float32)],
        compiler_params=pltpu.CompilerParams(
            dimension_semantics=("arbitrary",),
        ),
    )(table_t, cnt2, wt, b2)


def kernel(words, embedding, W, b):
    words3 = words.astype(jnp.int32).reshape(NS, PER_T // 128, 128)
    cnt = _count_sc(words3)
    table_t = embedding.T
    wt = W.T
    b2 = b.reshape(1, NTAGS)
    return _matvec_tc(table_t, cnt, wt, b2)


# speedup vs baseline: 5.2441x; 1.2686x over previous
import functools

import jax
import jax.numpy as jnp
from jax import lax
from jax.experimental import pallas as pl
from jax.experimental.pallas import tpu as pltpu
from jax.experimental.pallas import tpu_sc as plsc

NS = 16
LANES = 16

L = 16384
D = 64
NTAGS = 16
VOCAB = 1_000_000
CPAD = 1_048_576
PER_T = L // NS
CHUNK = CPAD // NS

BLK = 16384
NBLK = (VOCAB + BLK - 1) // BLK
SUBS = BLK // 128
TAIL_SUBS = (VOCAB - (NBLK - 1) * BLK + 127) // 128


def _count_sc(words3):
    mesh = plsc.VectorSubcoreMesh(
        core_axis_name="c", subcore_axis_name="s", num_cores=1
    )

    @functools.partial(
        pl.kernel,
        out_type=jax.ShapeDtypeStruct((CPAD,), jnp.float32),
        mesh=mesh,
        scratch_types=[
            pltpu.VMEM((PER_T // 128, 128), jnp.int32),
            pltpu.VMEM((CHUNK // 4,), jnp.float32),
            pltpu.VMEM((128,), jnp.float32),
            pltpu.VMEM_SHARED((CPAD,), jnp.float32),
        ],
    )
    def k(words_hbm, cnt_hbm, idx_v, zeros_v, ones_v, cnt_s):
        sid = lax.axis_index("s")

        z16 = jnp.zeros((LANES,), jnp.float32)
        o16 = jnp.ones((LANES,), jnp.float32)

        def zbody(i, _):
            zeros_v[pl.ds(i * LANES, LANES)] = z16
            return 0

        lax.fori_loop(0, (CHUNK // 4) // LANES, zbody, 0)
        for i in range(128 // LANES):
            ones_v[pl.ds(i * LANES, LANES)] = o16

        pltpu.sync_copy(words_hbm.at[sid], idx_v)
        for q in range(4):
            pltpu.sync_copy(
                zeros_v, cnt_s.at[pl.ds(sid * CHUNK + q * (CHUNK // 4),
                                        CHUNK // 4)]
            )
        plsc.subcore_barrier()

        for j in range(PER_T // 128):
            pltpu.sync_copy(ones_v, cnt_s.at[idx_v.at[j]], add=True)
        plsc.subcore_barrier()

        pltpu.sync_copy(
            cnt_s.at[pl.ds(sid * CHUNK, CHUNK)],
            cnt_hbm.at[pl.ds(sid * CHUNK, CHUNK)],
        )

    return k(words3)


def _matvec_tc(table_t, cnt2, wt, b2):
    def body(tbl_ref, cnt_ref, wt_ref, b_ref, out_ref, acc_ref):
        j = pl.program_id(0)

        @pl.when(j == 0)
        def _():
            acc_ref[...] = jnp.zeros((D, 128), jnp.float32)

        def accumulate(nsubs, mask_tail):
            acc = acc_ref[...]
            for sub in range(nsubs):
                t = tbl_ref[:, sub * 128:(sub + 1) * 128]
                if mask_tail:
                    base = (NBLK - 1) * BLK + sub * 128
                    ok = base + lax.iota(jnp.int32, 128) < VOCAB
                    t = jnp.where(ok[None, :], t, 0.0)
                row = cnt_ref[sub * 128:(sub + 1) * 128]
                acc = acc + t * row[None, :]
            acc_ref[...] = acc

        @pl.when(j < NBLK - 1)
        def _():
            accumulate(SUBS, False)

        @pl.when(j == NBLK - 1)
        def _():
            accumulate(TAIL_SUBS, True)
            emb = jnp.sum(acc_ref[...], axis=1)
            logits = jnp.sum(wt_ref[...] * emb[:, None], axis=0)
            out_ref[...] = logits[None, :] + b_ref[...]

        return

    return pl.pallas_call(
        body,
        grid=(NBLK,),
        in_specs=[
            pl.BlockSpec((D, BLK), lambda j: (0, j)),
            pl.BlockSpec((BLK,), lambda j: (j,)),
            pl.BlockSpec((D, NTAGS), lambda j: (0, 0)),
            pl.BlockSpec((1, NTAGS), lambda j: (0, 0)),
        ],
        out_specs=pl.BlockSpec((1, NTAGS), lambda j: (0, 0)),
        out_shape=jax.ShapeDtypeStruct((1, NTAGS), jnp.float32),
        scratch_shapes=[pltpu.VMEM((D, 128), jnp.float32)],
        compiler_params=pltpu.CompilerParams(
            dimension_semantics=("arbitrary",),
        ),
    )(table_t, cnt2, wt, b2)


def kernel(words, embedding, W, b):
    words3 = words.astype(jnp.int32).reshape(NS, PER_T // 128, 128)
    cnt = _count_sc(words3)
    table_t = embedding.T
    wt = W.T
    b2 = b.reshape(1, NTAGS)
    return _matvec_tc(table_t, cnt, wt, b2)


# speedup vs baseline: 5.5877x; 1.0655x over previous
import functools

import jax
import jax.numpy as jnp
from jax import lax
from jax.experimental import pallas as pl
from jax.experimental.pallas import tpu as pltpu
from jax.experimental.pallas import tpu_sc as plsc

NS = 16
LANES = 16

L = 16384
D = 64
NTAGS = 16
VOCAB = 1_000_000
CPAD = 1_048_576
PER_T = L // NS
CHUNK = CPAD // NS

BLK = 32768
NBLK = (VOCAB + BLK - 1) // BLK
SUBS = BLK // 128
TAIL_SUBS = (VOCAB - (NBLK - 1) * BLK + 127) // 128


def _count_sc(words3):
    mesh = plsc.VectorSubcoreMesh(
        core_axis_name="c", subcore_axis_name="s", num_cores=1
    )

    @functools.partial(
        pl.kernel,
        out_type=jax.ShapeDtypeStruct((CPAD,), jnp.float32),
        mesh=mesh,
        scratch_types=[
            pltpu.VMEM((PER_T // 128, 128), jnp.int32),
            pltpu.VMEM((CHUNK // 4,), jnp.float32),
            pltpu.VMEM((128,), jnp.float32),
            pltpu.VMEM_SHARED((CPAD,), jnp.float32),
        ],
    )
    def k(words_hbm, cnt_hbm, idx_v, zeros_v, ones_v, cnt_s):
        sid = lax.axis_index("s")

        z16 = jnp.zeros((LANES,), jnp.float32)
        o16 = jnp.ones((LANES,), jnp.float32)

        def zbody(i, _):
            zeros_v[pl.ds(i * LANES, LANES)] = z16
            return 0

        lax.fori_loop(0, (CHUNK // 4) // LANES, zbody, 0)
        for i in range(128 // LANES):
            ones_v[pl.ds(i * LANES, LANES)] = o16

        pltpu.sync_copy(words_hbm.at[sid], idx_v)
        for q in range(4):
            pltpu.sync_copy(
                zeros_v, cnt_s.at[pl.ds(sid * CHUNK + q * (CHUNK // 4),
                                        CHUNK // 4)]
            )
        plsc.subcore_barrier()

        for j in range(PER_T // 128):
            pltpu.sync_copy(ones_v, cnt_s.at[idx_v.at[j]], add=True)
        plsc.subcore_barrier()

        pltpu.sync_copy(
            cnt_s.at[pl.ds(sid * CHUNK, CHUNK)],
            cnt_hbm.at[pl.ds(sid * CHUNK, CHUNK)],
        )

    return k(words3)


def _matvec_tc(table_t, cnt2, wt, b2):
    def body(tbl_ref, cnt_ref, wt_ref, b_ref, out_ref, acc_ref):
        j = pl.program_id(0)

        @pl.when(j == 0)
        def _():
            acc_ref[...] = jnp.zeros((D, 128), jnp.float32)

        def accumulate(nsubs, mask_tail):
            acc = acc_ref[...]
            for sub in range(nsubs):
                t = tbl_ref[:, sub * 128:(sub + 1) * 128]
                if mask_tail:
                    base = (NBLK - 1) * BLK + sub * 128
                    ok = base + lax.iota(jnp.int32, 128) < VOCAB
                    t = jnp.where(ok[None, :], t, 0.0)
                row = cnt_ref[sub * 128:(sub + 1) * 128]
                acc = acc + t * row[None, :]
            acc_ref[...] = acc

        @pl.when(j < NBLK - 1)
        def _():
            accumulate(SUBS, False)

        @pl.when(j == NBLK - 1)
        def _():
            accumulate(TAIL_SUBS, True)
            emb = jnp.sum(acc_ref[...], axis=1)
            logits = jnp.sum(wt_ref[...] * emb[:, None], axis=0)
            out_ref[...] = logits[None, :] + b_ref[...]

        return

    return pl.pallas_call(
        body,
        grid=(NBLK,),
        in_specs=[
            pl.BlockSpec((D, BLK), lambda j: (0, j)),
            pl.BlockSpec((BLK,), lambda j: (j,)),
            pl.BlockSpec((D, NTAGS), lambda j: (0, 0)),
            pl.BlockSpec((1, NTAGS), lambda j: (0, 0)),
        ],
        out_specs=pl.BlockSpec((1, NTAGS), lambda j: (0, 0)),
        out_shape=jax.ShapeDtypeStruct((1, NTAGS), jnp.float32),
        scratch_shapes=[pltpu.VMEM((D, 128), jnp.float32)],
        compiler_params=pltpu.CompilerParams(
            dimension_semantics=("arbitrary",),
        ),
    )(table_t, cnt2, wt, b2)


def kernel(words, embedding, W, b):
    words3 = words.astype(jnp.int32).reshape(NS, PER_T // 128, 128)
    cnt = _count_sc(words3)
    table_t = embedding.T
    wt = W.T
    b2 = b.reshape(1, NTAGS)
    return _matvec_tc(table_t, cnt, wt, b2)
